# baseline (speedup 1.0000x reference)
import functools

import jax
import jax.numpy as jnp
from jax import lax
from jax.experimental import pallas as pl
from jax.experimental.pallas import tpu as pltpu
from jax.experimental.pallas import tpu_sc as plsc

V = 1024
D = 384
B = 131072
NC = 2
NS = 16
NW = NC * NS
BPW = B // NW
CHUNK = 64
NCH = BPW // CHUNK
NG = 3
NF = 2
GROUPS = D // 32


def _sc_gather(position_ids, table_bf16):
    mesh = plsc.VectorSubcoreMesh(core_axis_name="c", subcore_axis_name="s")

    @functools.partial(
        pl.kernel,
        mesh=mesh,
        out_type=jax.ShapeDtypeStruct((B, D), jnp.int32),
        compiler_params=pltpu.CompilerParams(use_tc_tiling_on_sc=False),
        scratch_types=[
            pltpu.VMEM((BPW,), jnp.int32),
            pltpu.VMEM((NG, CHUNK, D // 2), jnp.int32),
            pltpu.VMEM((NF, CHUNK, D), jnp.int32),
            pltpu.SemaphoreType.DMA,
            pltpu.SemaphoreType.DMA,
        ],
    )
    def k(ids_hbm, table_hbm, out_hbm, idx_v, braw, fout, gsem, wsem):
        wid = lax.axis_index("s") * NC + lax.axis_index("c")
        base = wid * BPW
        pltpu.sync_copy(ids_hbm.at[pl.ds(base, BPW)], idx_v)

        def start_gather(ch, bg):
            pltpu.async_copy(
                table_hbm.at[idx_v.at[pl.ds(ch * CHUNK, CHUNK)]],
                braw.at[bg],
                gsem,
            )

        def start_write(ch, bf):
            pltpu.async_copy(
                fout.at[bf],
                out_hbm.at[pl.ds(base + ch * CHUNK, CHUNK)],
                wsem,
            )

        def wait_gather():
            pltpu.make_async_copy(
                table_hbm.at[idx_v.at[pl.ds(0, CHUNK)]], braw.at[0], gsem
            ).wait()

        def wait_write():
            pltpu.make_async_copy(
                fout.at[0], out_hbm.at[pl.ds(base, CHUNK)], wsem
            ).wait()

        def convert(bg, bf):
            def row(r, c):
                for g in range(GROUPS):
                    w = braw[bg, r, pl.ds(g * 16, 16)]
                    a = w << 16
                    b = w & jnp.int32(-65536)
                    fout[bf, r, pl.ds(g * 32, 16)] = a
                    fout[bf, r, pl.ds(g * 32 + 16, 16)] = b
                return c

            lax.fori_loop(0, CHUNK, row, 0)

        start_gather(0, 0)
        start_gather(1, 1)
        for ch in (0, 1):
            wait_gather()
            start_gather(ch + 2, (ch + 2) % NG)
            convert(ch % NG, ch % NF)
            start_write(ch, ch % NF)

        def sextet(i, carry):
            ch0 = 6 * i + 2
            for p in range(6):
                ch = ch0 + p
                wait_gather()
                start_gather(ch + 2, (ch + 2) % NG)
                wait_write()
                convert((2 + p) % NG, p % NF)
                start_write(ch, p % NF)
            return carry

        lax.fori_loop(0, (NCH - 4) // 6, sextet, 0)

        for ch in (NCH - 2, NCH - 1):
            wait_gather()
            wait_write()
            convert(ch % NG, ch % NF)
            start_write(ch, ch % NF)
        wait_write()
        wait_write()

    return k(position_ids, table_bf16)


def kernel(position_ids, pos_embed):
    tb = pos_embed.astype(jnp.bfloat16)
    tb = tb.reshape(V, GROUPS, 2, 16).transpose(0, 1, 3, 2).reshape(V, D // 2, 2)
    tb = jax.lax.bitcast_convert_type(tb, jnp.int32)
    out = _sc_gather(position_ids.astype(jnp.int32), tb)
    return jax.lax.bitcast_convert_type(out, jnp.float32)


# speedup vs baseline: 1.2191x; 1.2191x over previous
import functools

import jax
import jax.numpy as jnp
from jax import lax
from jax.experimental import pallas as pl
from jax.experimental.pallas import tpu as pltpu
from jax.experimental.pallas import tpu_sc as plsc

V = 1024
D = 384
B = 131072
NC = 2
NS = 16
NW = NC * NS
BPW = B // NW
CHUNK = 64
NCH = BPW // CHUNK
NG = 3
NF = 2
GROUPS = D // 32


def _sc_gather(position_ids, table_bf16):
    mesh = plsc.VectorSubcoreMesh(core_axis_name="c", subcore_axis_name="s")

    @functools.partial(
        pl.kernel,
        mesh=mesh,
        out_type=jax.ShapeDtypeStruct((B, D), jnp.int32),
        compiler_params=pltpu.CompilerParams(use_tc_tiling_on_sc=False),
        scratch_types=[
            pltpu.VMEM((BPW,), jnp.int32),
            pltpu.VMEM((NG, CHUNK, D // 2), jnp.int32),
            pltpu.VMEM((NF, CHUNK, D), jnp.int32),
            pltpu.SemaphoreType.DMA,
            pltpu.SemaphoreType.DMA,
        ],
    )
    def k(ids_hbm, table_hbm, out_hbm, idx_v, braw, fout, gsem, wsem):
        wid = lax.axis_index("s") * NC + lax.axis_index("c")
        base = wid * BPW
        pltpu.sync_copy(ids_hbm.at[pl.ds(base, BPW)], idx_v)

        def start_gather(ch, bg):
            pltpu.async_copy(
                table_hbm.at[idx_v.at[pl.ds(ch * CHUNK, CHUNK)]],
                braw.at[bg],
                gsem,
            )

        def start_write(ch, bf):
            pltpu.async_copy(
                fout.at[bf],
                out_hbm.at[pl.ds(base + ch * CHUNK, CHUNK)],
                wsem,
            )

        def wait_gather():
            pltpu.make_async_copy(
                table_hbm.at[idx_v.at[pl.ds(0, CHUNK)]], braw.at[0], gsem
            ).wait()

        def wait_write():
            pltpu.make_async_copy(
                fout.at[0], out_hbm.at[pl.ds(base, CHUNK)], wsem
            ).wait()

        def convert(bg, bf):
            @plsc.parallel_loop(0, CHUNK, unroll=4)
            def row(r):
                for g in range(GROUPS):
                    w = braw[bg, r, pl.ds(g * 16, 16)]
                    a = w << 16
                    b = w & jnp.int32(-65536)
                    fout[bf, r, pl.ds(g * 32, 16)] = a
                    fout[bf, r, pl.ds(g * 32 + 16, 16)] = b

        start_gather(0, 0)
        start_gather(1, 1)
        for ch in (0, 1):
            wait_gather()
            start_gather(ch + 2, (ch + 2) % NG)
            convert(ch % NG, ch % NF)
            start_write(ch, ch % NF)

        def sextet(i, carry):
            ch0 = 6 * i + 2
            for p in range(6):
                ch = ch0 + p
                wait_gather()
                start_gather(ch + 2, (ch + 2) % NG)
                wait_write()
                convert((2 + p) % NG, p % NF)
                start_write(ch, p % NF)
            return carry

        lax.fori_loop(0, (NCH - 4) // 6, sextet, 0)

        for ch in (NCH - 2, NCH - 1):
            wait_gather()
            wait_write()
            convert(ch % NG, ch % NF)
            start_write(ch, ch % NF)
        wait_write()
        wait_write()

    return k(position_ids, table_bf16)


def kernel(position_ids, pos_embed):
    tb = pos_embed.astype(jnp.bfloat16)
    tb = tb.reshape(V, GROUPS, 2, 16).transpose(0, 1, 3, 2).reshape(V, D // 2, 2)
    tb = jax.lax.bitcast_convert_type(tb, jnp.int32)
    out = _sc_gather(position_ids.astype(jnp.int32), tb)
    return jax.lax.bitcast_convert_type(out, jnp.float32)


# speedup vs baseline: 1.9726x; 1.6181x over previous
import functools

import jax
import jax.numpy as jnp
from jax import lax
from jax.experimental import pallas as pl
from jax.experimental.pallas import tpu as pltpu
from jax.experimental.pallas import tpu_sc as plsc

V = 1024
D = 384
B = 131072
NC = 2
NS = 16
NW = NC * NS
BPW = B // NW
CHUNK = 64
NCH = BPW // CHUNK
NG = 3
NF = 2
GROUPS = D // 32
PW = 256


def _sc_gather(position_ids, table_bf16):
    mesh = plsc.VectorSubcoreMesh(core_axis_name="c", subcore_axis_name="s")

    @functools.partial(
        pl.kernel,
        mesh=mesh,
        out_type=jax.ShapeDtypeStruct((B, D), jnp.int32),
        scratch_types=[
            pltpu.VMEM((BPW,), jnp.int32),
            pltpu.VMEM((NG, CHUNK, PW), jnp.int32),
            pltpu.VMEM((NF, CHUNK, D), jnp.int32),
            pltpu.SemaphoreType.DMA,
            pltpu.SemaphoreType.DMA,
        ],
    )
    def k(ids_hbm, table_hbm, out_hbm, idx_v, braw, fout, gsem, wsem):
        wid = lax.axis_index("s") * NC + lax.axis_index("c")
        base = wid * BPW
        pltpu.sync_copy(ids_hbm.at[pl.ds(base, BPW)], idx_v)

        def start_gather(ch, bg):
            pltpu.async_copy(
                table_hbm.at[idx_v.at[pl.ds(ch * CHUNK, CHUNK)]],
                braw.at[bg],
                gsem,
            )

        def start_write(ch, bf):
            pltpu.async_copy(
                fout.at[bf],
                out_hbm.at[pl.ds(base + ch * CHUNK, CHUNK)],
                wsem,
            )

        def wait_gather():
            pltpu.make_async_copy(
                table_hbm.at[idx_v.at[pl.ds(0, CHUNK)]], braw.at[0], gsem
            ).wait()

        def wait_write():
            pltpu.make_async_copy(
                fout.at[0], out_hbm.at[pl.ds(base, CHUNK)], wsem
            ).wait()

        def convert(bg, bf):
            @plsc.parallel_loop(0, CHUNK, unroll=4)
            def row(r):
                for g in range(GROUPS):
                    w = braw[bg, r, pl.ds(g * 16, 16)]
                    a = w << 16
                    b = w & jnp.int32(-65536)
                    fout[bf, r, pl.ds(g * 32, 16)] = a
                    fout[bf, r, pl.ds(g * 32 + 16, 16)] = b

        start_gather(0, 0)
        start_gather(1, 1)
        for ch in (0, 1):
            wait_gather()
            start_gather(ch + 2, (ch + 2) % NG)
            convert(ch % NG, ch % NF)
            start_write(ch, ch % NF)

        def sextet(i, carry):
            ch0 = 6 * i + 2
            for p in range(6):
                ch = ch0 + p
                wait_gather()
                start_gather(ch + 2, (ch + 2) % NG)
                wait_write()
                convert((2 + p) % NG, p % NF)
                start_write(ch, p % NF)
            return carry

        lax.fori_loop(0, (NCH - 4) // 6, sextet, 0)

        for ch in (NCH - 2, NCH - 1):
            wait_gather()
            wait_write()
            convert(ch % NG, ch % NF)
            start_write(ch, ch % NF)
        wait_write()
        wait_write()

    return k(position_ids, table_bf16)


def kernel(position_ids, pos_embed):
    tb = pos_embed.astype(jnp.bfloat16)
    tb = tb.reshape(V, GROUPS, 2, 16).transpose(0, 1, 3, 2).reshape(V, D // 2, 2)
    tb = jax.lax.bitcast_convert_type(tb, jnp.int32)
    tb = jnp.pad(tb, ((0, 0), (0, PW - D // 2)))
    out = _sc_gather(position_ids.astype(jnp.int32), tb)
    return jax.lax.bitcast_convert_type(out, jnp.float32)


# speedup vs baseline: 3.5467x; 1.7980x over previous
import functools

import jax
import jax.numpy as jnp
from jax import lax
from jax.experimental import pallas as pl
from jax.experimental.pallas import tpu as pltpu
from jax.experimental.pallas import tpu_sc as plsc

V = 1024
D = 384
B = 131072
NC = 2
NS = 16
NW = NC * NS
BPW = B // NW
CHUNK = 64
NCH = BPW // CHUNK
NG = 3
NF = 2
GROUPS = D // 32
PW = 256


def _sc_gather(position_ids, table_bf16):
    mesh = plsc.VectorSubcoreMesh(core_axis_name="c", subcore_axis_name="s")

    @functools.partial(
        pl.kernel,
        mesh=mesh,
        out_type=jax.ShapeDtypeStruct((B, D), jnp.float32),
        compiler_params=pltpu.CompilerParams(needs_layout_passes=False),
        scratch_types=[
            pltpu.VMEM((BPW,), jnp.int32),
            pltpu.VMEM((NG, CHUNK, PW), jnp.int32),
            pltpu.VMEM((NF, CHUNK, D), jnp.float32),
            pltpu.SemaphoreType.DMA,
            pltpu.SemaphoreType.DMA,
        ],
    )
    def k(ids_hbm, table_hbm, out_hbm, idx_v, braw, fout, gsem, wsem):
        wid = lax.axis_index("s") * NC + lax.axis_index("c")
        base = wid * BPW
        pltpu.sync_copy(ids_hbm.at[pl.ds(base, BPW)], idx_v)

        def start_gather(ch, bg):
            pltpu.async_copy(
                table_hbm.at[idx_v.at[pl.ds(ch * CHUNK, CHUNK)]],
                braw.at[bg],
                gsem,
            )

        def start_write(ch, bf):
            pltpu.async_copy(
                fout.at[bf],
                out_hbm.at[pl.ds(base + ch * CHUNK, CHUNK)],
                wsem,
            )

        def wait_gather():
            pltpu.make_async_copy(
                table_hbm.at[idx_v.at[pl.ds(0, CHUNK)]], braw.at[0], gsem
            ).wait()

        def wait_write():
            pltpu.make_async_copy(
                fout.at[0], out_hbm.at[pl.ds(base, CHUNK)], wsem
            ).wait()

        def convert(bg, bf):
            @plsc.parallel_loop(0, CHUNK, unroll=4)
            def row(r):
                for g in range(GROUPS):
                    w = braw[bg, r, pl.ds(g * 16, 16)]
                    a = plsc.bitcast(w << 16, jnp.float32)
                    b = plsc.bitcast(w & jnp.int32(-65536), jnp.float32)
                    fout[bf, r, pl.ds(g * 32, 16)] = a
                    fout[bf, r, pl.ds(g * 32 + 16, 16)] = b

        start_gather(0, 0)
        start_gather(1, 1)
        for ch in (0, 1):
            wait_gather()
            start_gather(ch + 2, (ch + 2) % NG)
            convert(ch % NG, ch % NF)
            start_write(ch, ch % NF)

        def sextet(i, carry):
            ch0 = 6 * i + 2
            for p in range(6):
                ch = ch0 + p
                wait_gather()
                start_gather(ch + 2, (ch + 2) % NG)
                wait_write()
                convert((2 + p) % NG, p % NF)
                start_write(ch, p % NF)
            return carry

        lax.fori_loop(0, (NCH - 4) // 6, sextet, 0)

        for ch in (NCH - 2, NCH - 1):
            wait_gather()
            wait_write()
            convert(ch % NG, ch % NF)
            start_write(ch, ch % NF)
        wait_write()
        wait_write()

    return k(position_ids, table_bf16)


def kernel(position_ids, pos_embed):
    tb = pos_embed.astype(jnp.bfloat16)
    tb = tb.reshape(V, GROUPS, 2, 16).transpose(0, 1, 3, 2).reshape(V, D // 2, 2)
    tb = jax.lax.bitcast_convert_type(tb, jnp.int32)
    tb = jnp.pad(tb, ((0, 0), (0, PW - D // 2)))
    return _sc_gather(position_ids.astype(jnp.int32), tb)
